# baseline (speedup 1.0000x reference)
import functools

import jax
import jax.numpy as jnp
from jax import lax
from jax.experimental import pallas as pl
from jax.experimental.pallas import tpu as pltpu
from jax.experimental.pallas import tpu_sc as plsc

_F32 = jnp.float32
_NC, _NS = 2, 16
_NW = _NC * _NS
_C = 100
_SUB = 2
_CC = _C * _SUB


def _silu(x):
    return x * jax.nn.sigmoid(x)


def _tc_tables(V2, WA, WB):
    N = V2.shape[0]

    def body(v_ref, wa_ref, wb_ref, a_ref, b_ref):
        v = v_ref[...]
        a_ref[...] = jnp.dot(v, wa_ref[...], preferred_element_type=_F32)
        b_ref[...] = jnp.dot(v, wb_ref[...], preferred_element_type=_F32)

    return pl.pallas_call(
        body,
        out_shape=[
            jax.ShapeDtypeStruct((N, 64), _F32),
            jax.ShapeDtypeStruct((N, 64), _F32),
        ],
    )(V2, WA, WB)


def _sc_gather(A, B, src2d, dst2d, M):
    D = A.shape[1]
    per_w = M // _NW
    n_ch = per_w // _CC
    rounds = n_ch // 2
    mesh = plsc.VectorSubcoreMesh(core_axis_name="c", subcore_axis_name="s")

    @functools.partial(
        pl.kernel,
        mesh=mesh,
        out_type=jax.ShapeDtypeStruct((M, 2 * D), _F32),
        scratch_types=[
            pltpu.VMEM((_SUB, _C), jnp.int32),
            pltpu.VMEM((_SUB, _C), jnp.int32),
            pltpu.VMEM((_SUB, _C), jnp.int32),
            pltpu.VMEM((_SUB, _C), jnp.int32),
            pltpu.VMEM((_CC, D), _F32),
            pltpu.VMEM((_CC, D), _F32),
            pltpu.VMEM((_CC, D), _F32),
            pltpu.VMEM((_CC, D), _F32),
            pltpu.SemaphoreType.DMA,
            pltpu.SemaphoreType.DMA,
            pltpu.SemaphoreType.DMA,
            pltpu.SemaphoreType.DMA,
            pltpu.SemaphoreType.DMA,
            pltpu.SemaphoreType.DMA,
        ],
        compiler_params=pltpu.CompilerParams(use_tc_tiling_on_sc=False),
    )
    def k(a_hbm, b_hbm, src_hbm, dst_hbm, outp, idxs0, idxs1, idxd0, idxd1,
          ga0, ga1, gb0, gb1, sg0, sg1, sw0, sw1, si0, si1):
        idxs = [idxs0, idxs1]
        idxd = [idxd0, idxd1]
        ga = [ga0, ga1]
        gb = [gb0, gb1]
        sem_g = [sg0, sg1]
        sem_w = [sw0, sw1]
        sem_i = [si0, si1]

        wid = lax.axis_index("s") * _NC + lax.axis_index("c")
        base = wid * per_w
        idx_base = wid * (per_w // _C)

        def fire_idx(c, b):
            r0 = idx_base + c * _SUB
            pltpu.async_copy(src_hbm.at[pl.ds(r0, _SUB)], idxs[b], sem_i[b])
            pltpu.async_copy(dst_hbm.at[pl.ds(r0, _SUB)], idxd[b], sem_i[b])

        def wait_idx(b):
            pltpu.make_async_copy(src_hbm.at[pl.ds(0, _SUB)], idxs[b],
                                  sem_i[b]).wait()
            pltpu.make_async_copy(dst_hbm.at[pl.ds(0, _SUB)], idxd[b],
                                  sem_i[b]).wait()

        def fire_gather(b):
            for j in range(_SUB):
                pltpu.async_copy(a_hbm.at[idxs[b].at[j]],
                                 ga[b].at[pl.ds(j * _C, _C)], sem_g[b])
                pltpu.async_copy(b_hbm.at[idxd[b].at[j]],
                                 gb[b].at[pl.ds(j * _C, _C)], sem_g[b])

        def wait_gather(b):
            for j in range(_SUB):
                pltpu.make_async_copy(a_hbm.at[idxs[b].at[j]],
                                      ga[b].at[pl.ds(j * _C, _C)],
                                      sem_g[b]).wait()
                pltpu.make_async_copy(b_hbm.at[idxd[b].at[j]],
                                      gb[b].at[pl.ds(j * _C, _C)],
                                      sem_g[b]).wait()

        def fire_write(c, b):
            sl = pl.ds(base + c * _CC, _CC)
            pltpu.async_copy(ga[b], outp.at[sl, pl.ds(0, D)], sem_w[b])
            pltpu.async_copy(gb[b], outp.at[sl, pl.ds(D, D)], sem_w[b])

        def wait_write(b):
            sl = pl.ds(0, _CC)
            pltpu.make_async_copy(ga[b], outp.at[sl, pl.ds(0, D)],
                                  sem_w[b]).wait()
            pltpu.make_async_copy(gb[b], outp.at[sl, pl.ds(D, D)],
                                  sem_w[b]).wait()

        for b in range(2):
            fire_idx(b, b)
        for b in range(2):
            wait_idx(b)
            fire_gather(b)

        for b in range(2):
            wait_gather(b)
            fire_write(b, b)
            fire_idx(b + 2, b)
        for b in range(2):
            wait_idx(b)
            wait_write(b)
            fire_gather(b)

        def round_body(r, carry):
            for b in range(2):
                c = 2 * r + b
                wait_gather(b)
                fire_write(c, b)
                fire_idx(c + 2, b)
            for b in range(2):
                wait_idx(b)
                wait_write(b)
                fire_gather(b)
            return carry

        lax.fori_loop(1, rounds - 1, round_body, 0)

        for b in range(2):
            wait_gather(b)
            fire_write(n_ch - 2 + b, b)
        for b in range(2):
            wait_write(b)

    return k(A, B, src2d, dst2d)


def _tc_edges(P, E2, WE, bcat, W2, Wa1, ba1, Wa2, ba2, b2, Wu2, bu2):
    M = E2.shape[0]
    Mb = 4000
    grid = (M // Mb,)

    def body(p_ref, e_ref, we_ref, bc_ref, w2_ref, wa1_ref, ba1_ref,
             wa2_ref, ba2_ref, b2_ref, wu2_ref, bu2_ref, delta_ref, g_ref):
        pp = p_ref[...]
        h = (pp[:, :64] + pp[:, 64:]
             + jnp.dot(e_ref[...], we_ref[...], preferred_element_type=_F32)
             + bc_ref[...])
        s = _silu(h)
        s1 = s[:, :32]
        su = s[:, 32:]
        w2a = jnp.dot(w2_ref[...], wa1_ref[...], preferred_element_type=_F32)
        ba1p = ba1_ref[...] + jnp.dot(b2_ref[...], wa1_ref[...],
                                      preferred_element_type=_F32)
        t = _silu(jnp.dot(s1, w2a, preferred_element_type=_F32) + ba1p)
        logit = jnp.dot(t, wa2_ref[...], preferred_element_type=_F32) + ba2_ref[...]
        w = jnp.exp(jnp.clip(logit, -30.0, 30.0))
        g_ref[...] = jnp.concatenate(
            [w * s1, jnp.broadcast_to(w, (Mb, 16)),
             jnp.zeros((Mb, 80), _F32)], axis=1)
        delta_ref[...] = (jnp.dot(su, wu2_ref[...], preferred_element_type=_F32)
                          + bu2_ref[...])

    full = lambda shape: pl.BlockSpec(shape, lambda i: (0, 0))
    return pl.pallas_call(
        body,
        grid=grid,
        in_specs=[
            pl.BlockSpec((Mb, 128), lambda i: (i, 0)),
            pl.BlockSpec((Mb, 128), lambda i: (i, 0)),
            full((128, 64)),
            full((1, 64)),
            full((32, 128)),
            full((128, 8)),
            full((1, 8)),
            full((8, 1)),
            full((1, 1)),
            full((1, 128)),
            full((32, 128)),
            full((1, 128)),
        ],
        out_specs=[
            pl.BlockSpec((Mb, 128), lambda i: (i, 0)),
            pl.BlockSpec((Mb, 128), lambda i: (i, 0)),
        ],
        out_shape=[
            jax.ShapeDtypeStruct((M, 128), _F32),
            jax.ShapeDtypeStruct((M, 128), _F32),
        ],
        compiler_params=pltpu.CompilerParams(
            dimension_semantics=("arbitrary",)),
    )(P, E2, WE, bcat, W2, Wa1, ba1, Wa2, ba2, b2, Wu2, bu2)


_SC = 40
_SSUB = _CC // _SC


def _sc_scatter(g, dst1d, zrows):
    M = g.shape[0]
    D = zrows.shape[1]
    Np = zrows.shape[0]
    per_w = M // _NW
    n_ch = per_w // _CC
    rounds = n_ch // 2
    rpt = Np // _NS
    mesh = plsc.VectorSubcoreMesh(core_axis_name="c", subcore_axis_name="s")

    @functools.partial(
        pl.kernel,
        mesh=mesh,
        out_type=jax.ShapeDtypeStruct((2 * Np, D), _F32),
        scratch_types=(
            [pltpu.VMEM((_SC,), jnp.int32) for _ in range(2 * _SSUB)]
            + [
                pltpu.VMEM((_CC, D), _F32),
                pltpu.VMEM((_CC, D), _F32),
                pltpu.VMEM_SHARED((Np, D), _F32),
                pltpu.SemaphoreType.DMA,
                pltpu.SemaphoreType.DMA,
                pltpu.SemaphoreType.DMA,
                pltpu.SemaphoreType.DMA,
            ]
        ),
        compiler_params=pltpu.CompilerParams(use_tc_tiling_on_sc=False),
    )
    def k(g_hbm, dst_hbm, z_hbm, out, *rest):
        idx = [list(rest[:_SSUB]), list(rest[_SSUB:2 * _SSUB])]
        r0, r1, acc, sin0, sin1, ssc0, ssc1 = rest[2 * _SSUB:]
        rows = [r0, r1]
        sem_in = [sin0, sin1]
        sem_sc = [ssc0, ssc1]

        cid = lax.axis_index("c")
        sid = lax.axis_index("s")
        wid = sid * _NC + cid
        base = wid * per_w

        pltpu.sync_copy(z_hbm.at[pl.ds(sid * rpt, rpt)],
                        acc.at[pl.ds(sid * rpt, rpt)])
        plsc.subcore_barrier()

        def fire_in(c, b):
            off = base + c * _CC
            for j in range(_SSUB):
                pltpu.async_copy(dst_hbm.at[pl.ds(off + j * _SC, _SC)],
                                 idx[b][j], sem_in[b])
            pltpu.async_copy(g_hbm.at[pl.ds(off, _CC), pl.ds(0, D)], rows[b],
                             sem_in[b])

        def wait_in(b):
            for j in range(_SSUB):
                pltpu.make_async_copy(dst_hbm.at[pl.ds(0, _SC)], idx[b][j],
                                      sem_in[b]).wait()
            pltpu.make_async_copy(g_hbm.at[pl.ds(0, _CC), pl.ds(0, D)],
                                  rows[b], sem_in[b]).wait()

        def fire_scatter(b):
            for j in range(_SSUB):
                pltpu.async_copy(rows[b].at[pl.ds(j * _SC, _SC)],
                                 acc.at[idx[b][j]], sem_sc[b], add=True)

        def wait_scatter(b):
            for j in range(_SSUB):
                pltpu.make_async_copy(rows[b].at[pl.ds(j * _SC, _SC)],
                                      acc.at[idx[b][j]], sem_sc[b]).wait()

        for b in range(2):
            fire_in(b, b)

        def round_body(r, carry):
            for b in range(2):
                wait_in(b)
                fire_scatter(b)
            for b in range(2):
                wait_scatter(b)
                fire_in(2 * r + b + 2, b)
            return carry

        lax.fori_loop(0, rounds - 1, round_body, 0)

        for b in range(2):
            wait_in(b)
            fire_scatter(b)
        for b in range(2):
            wait_scatter(b)

        plsc.subcore_barrier()
        pltpu.sync_copy(acc.at[pl.ds(sid * rpt, rpt)],
                        out.at[pl.ds(cid * Np + sid * rpt, rpt)])

    return k(g, dst1d, zrows)


def _tc_nodes(V2, G0, G1, G2, G3, Wn1, bn1, Wn2, bn2, W2, b2):
    N = V2.shape[0]

    def body(v_ref, g0_ref, g1_ref, g2_ref, g3_ref, wn1_ref, bn1_ref,
             wn2_ref, bn2_ref, w2_ref, b2_ref, out_ref):
        gs = g0_ref[...] + g1_ref[...] + g2_ref[...] + g3_ref[...]
        Gm = gs[:, :32]
        dn = gs[:, 32:33]
        u = Gm / (dn + 1e-16)
        wn1a = wn1_ref[:128, :]
        wn1b = wn1_ref[128:, :]
        w2n = jnp.dot(w2_ref[...], wn1b, preferred_element_type=_F32)
        bw = jnp.dot(b2_ref[...], wn1b, preferred_element_type=_F32)
        pre = (jnp.dot(v_ref[...], wn1a, preferred_element_type=_F32)
               + jnp.dot(u, w2n, preferred_element_type=_F32)
               + bn1_ref[...]
               + jnp.where(dn > 0, bw, 0.0))
        out_ref[...] = (jnp.dot(_silu(pre), wn2_ref[...],
                                preferred_element_type=_F32) + bn2_ref[...])

    return pl.pallas_call(
        body,
        out_shape=jax.ShapeDtypeStruct((N, 128), _F32),
    )(V2, G0, G1, G2, G3, Wn1, bn1, Wn2, bn2, W2, b2)


def kernel(V, E, edges, W1, b1, W2, b2, Wa1, ba1, Wa2, ba2, Wu1, bu1, Wu2, bu2,
           Wn1, bn1, Wn2, bn2):
    V2 = V[0]
    E2 = E[0]
    M = E2.shape[0]
    src2d = edges[0, :, 0].reshape(M // _C, _C)
    dst1d = edges[0, :, 1]
    dst2d = dst1d.reshape(M // _C, _C)
    N = V2.shape[0]

    WA = jnp.concatenate([W1[:128], Wu1[:128]], axis=1)
    WB = jnp.concatenate([W1[128:256], Wu1[128:256]], axis=1)
    WE = jnp.concatenate([W1[256:], Wu1[256:]], axis=1)
    bcat = jnp.concatenate([b1, bu1])[None, :]

    A, B = _tc_tables(V2, WA, WB)

    Mh0 = 192000
    Np = 16 * 640
    zrows = jnp.zeros((Np, 48), _F32)
    halves = []
    for lo, hi in ((0, Mh0), (Mh0, M)):
        Mh = hi - lo
        P = _sc_gather(A, B, src2d[lo // _C:hi // _C],
                       dst2d[lo // _C:hi // _C], Mh)
        delta, g = _tc_edges(P, E2[lo:hi], WE, bcat, W2, Wa1, ba1[None, :],
                             Wa2, ba2[None, :], b2[None, :], Wu2,
                             bu2[None, :])
        Gp = _sc_scatter(g, dst1d[lo:hi], zrows)
        halves.append((delta, Gp))

    delta = jnp.concatenate([halves[0][0], halves[1][0]], axis=0)
    node_out = _tc_nodes(V2, halves[0][1][:N], halves[0][1][Np:Np + N],
                         halves[1][1][:N], halves[1][1][Np:Np + N],
                         Wn1, bn1[None, :], Wn2, bn2[None, :], W2,
                         b2[None, :])
    return node_out[None], delta[None]


# speedup vs baseline: 1.3298x; 1.3298x over previous
import functools

import jax
import jax.numpy as jnp
from jax import lax
from jax.experimental import pallas as pl
from jax.experimental.pallas import tpu as pltpu
from jax.experimental.pallas import tpu_sc as plsc

_F32 = jnp.float32
_NC, _NS = 2, 16
_NW = _NC * _NS
_C = 100
_SUB = 2
_CC = _C * _SUB


def _silu(x):
    return x * jax.nn.sigmoid(x)


def _tc_tables(V2, WA, WB):
    N = V2.shape[0]

    def body(v_ref, wa_ref, wb_ref, a_ref, b_ref):
        v = v_ref[...]
        a_ref[...] = jnp.dot(v, wa_ref[...], preferred_element_type=_F32)
        b_ref[...] = jnp.dot(v, wb_ref[...], preferred_element_type=_F32)

    return pl.pallas_call(
        body,
        out_shape=[
            jax.ShapeDtypeStruct((N, 64), _F32),
            jax.ShapeDtypeStruct((N, 64), _F32),
        ],
    )(V2, WA, WB)


def _sc_gather(A, B, src2d, dst2d, M):
    D = A.shape[1]
    per_w = M // _NW
    n_ch = per_w // _CC
    rounds = n_ch // 2
    mesh = plsc.VectorSubcoreMesh(core_axis_name="c", subcore_axis_name="s")

    @functools.partial(
        pl.kernel,
        mesh=mesh,
        out_type=jax.ShapeDtypeStruct((M, 2 * D), _F32),
        scratch_types=[
            pltpu.VMEM((_SUB, _C), jnp.int32),
            pltpu.VMEM((_SUB, _C), jnp.int32),
            pltpu.VMEM((_SUB, _C), jnp.int32),
            pltpu.VMEM((_SUB, _C), jnp.int32),
            pltpu.VMEM((_CC, D), _F32),
            pltpu.VMEM((_CC, D), _F32),
            pltpu.VMEM((_CC, D), _F32),
            pltpu.VMEM((_CC, D), _F32),
            pltpu.SemaphoreType.DMA,
            pltpu.SemaphoreType.DMA,
            pltpu.SemaphoreType.DMA,
            pltpu.SemaphoreType.DMA,
            pltpu.SemaphoreType.DMA,
            pltpu.SemaphoreType.DMA,
        ],
        compiler_params=pltpu.CompilerParams(use_tc_tiling_on_sc=False),
    )
    def k(a_hbm, b_hbm, src_hbm, dst_hbm, outp, idxs0, idxs1, idxd0, idxd1,
          ga0, ga1, gb0, gb1, sg0, sg1, sw0, sw1, si0, si1):
        idxs = [idxs0, idxs1]
        idxd = [idxd0, idxd1]
        ga = [ga0, ga1]
        gb = [gb0, gb1]
        sem_g = [sg0, sg1]
        sem_w = [sw0, sw1]
        sem_i = [si0, si1]

        wid = lax.axis_index("s") * _NC + lax.axis_index("c")
        base = wid * per_w
        idx_base = wid * (per_w // _C)

        def fire_idx(c, b):
            r0 = idx_base + c * _SUB
            pltpu.async_copy(src_hbm.at[pl.ds(r0, _SUB)], idxs[b], sem_i[b])
            pltpu.async_copy(dst_hbm.at[pl.ds(r0, _SUB)], idxd[b], sem_i[b])

        def wait_idx(b):
            pltpu.make_async_copy(src_hbm.at[pl.ds(0, _SUB)], idxs[b],
                                  sem_i[b]).wait()
            pltpu.make_async_copy(dst_hbm.at[pl.ds(0, _SUB)], idxd[b],
                                  sem_i[b]).wait()

        def fire_gather(b):
            for j in range(_SUB):
                pltpu.async_copy(a_hbm.at[idxs[b].at[j]],
                                 ga[b].at[pl.ds(j * _C, _C)], sem_g[b])
                pltpu.async_copy(b_hbm.at[idxd[b].at[j]],
                                 gb[b].at[pl.ds(j * _C, _C)], sem_g[b])

        def wait_gather(b):
            for j in range(_SUB):
                pltpu.make_async_copy(a_hbm.at[idxs[b].at[j]],
                                      ga[b].at[pl.ds(j * _C, _C)],
                                      sem_g[b]).wait()
                pltpu.make_async_copy(b_hbm.at[idxd[b].at[j]],
                                      gb[b].at[pl.ds(j * _C, _C)],
                                      sem_g[b]).wait()

        def fire_write(c, b):
            sl = pl.ds(base + c * _CC, _CC)
            pltpu.async_copy(ga[b], outp.at[sl, pl.ds(0, D)], sem_w[b])
            pltpu.async_copy(gb[b], outp.at[sl, pl.ds(D, D)], sem_w[b])

        def wait_write(b):
            sl = pl.ds(0, _CC)
            pltpu.make_async_copy(ga[b], outp.at[sl, pl.ds(0, D)],
                                  sem_w[b]).wait()
            pltpu.make_async_copy(gb[b], outp.at[sl, pl.ds(D, D)],
                                  sem_w[b]).wait()

        for b in range(2):
            fire_idx(b, b)
        for b in range(2):
            wait_idx(b)
            fire_gather(b)

        for b in range(2):
            wait_gather(b)
            fire_write(b, b)
            fire_idx(b + 2, b)
        for b in range(2):
            wait_idx(b)
            wait_write(b)
            fire_gather(b)

        def round_body(r, carry):
            for b in range(2):
                c = 2 * r + b
                wait_gather(b)
                fire_write(c, b)
                fire_idx(c + 2, b)
            for b in range(2):
                wait_idx(b)
                wait_write(b)
                fire_gather(b)
            return carry

        lax.fori_loop(1, rounds - 1, round_body, 0)

        for b in range(2):
            wait_gather(b)
            fire_write(n_ch - 2 + b, b)
        for b in range(2):
            wait_write(b)

    return k(A, B, src2d, dst2d)


def _tc_edges(P, E2, WE, bcat, W2, Wa1, ba1, Wa2, ba2, b2, Wu2, bu2):
    M = E2.shape[0]
    Mb = 4000
    grid = (M // Mb,)

    def body(p_ref, e_ref, we_ref, bc_ref, w2_ref, wa1_ref, ba1_ref,
             wa2_ref, ba2_ref, b2_ref, wu2_ref, bu2_ref, delta_ref, g_ref):
        pp = p_ref[...]
        h = (pp[:, :64] + pp[:, 64:]
             + jnp.dot(e_ref[...], we_ref[...], preferred_element_type=_F32)
             + bc_ref[...])
        s = _silu(h)
        s1 = s[:, :32]
        su = s[:, 32:]
        w2a = jnp.dot(w2_ref[...], wa1_ref[...], preferred_element_type=_F32)
        ba1p = ba1_ref[...] + jnp.dot(b2_ref[...], wa1_ref[...],
                                      preferred_element_type=_F32)
        t = _silu(jnp.dot(s1, w2a, preferred_element_type=_F32) + ba1p)
        logit = jnp.dot(t, wa2_ref[...], preferred_element_type=_F32) + ba2_ref[...]
        w = jnp.exp(jnp.clip(logit, -30.0, 30.0))
        g_ref[...] = jnp.concatenate(
            [w * s1, jnp.broadcast_to(w, (Mb, 16)),
             jnp.zeros((Mb, 80), _F32)], axis=1)
        delta_ref[...] = (jnp.dot(su, wu2_ref[...], preferred_element_type=_F32)
                          + bu2_ref[...])

    full = lambda shape: pl.BlockSpec(shape, lambda i: (0, 0))
    return pl.pallas_call(
        body,
        grid=grid,
        in_specs=[
            pl.BlockSpec((Mb, 128), lambda i: (i, 0)),
            pl.BlockSpec((Mb, 128), lambda i: (i, 0)),
            full((128, 64)),
            full((1, 64)),
            full((32, 128)),
            full((128, 8)),
            full((1, 8)),
            full((8, 1)),
            full((1, 1)),
            full((1, 128)),
            full((32, 128)),
            full((1, 128)),
        ],
        out_specs=[
            pl.BlockSpec((Mb, 128), lambda i: (i, 0)),
            pl.BlockSpec((Mb, 128), lambda i: (i, 0)),
        ],
        out_shape=[
            jax.ShapeDtypeStruct((M, 128), _F32),
            jax.ShapeDtypeStruct((M, 128), _F32),
        ],
        compiler_params=pltpu.CompilerParams(
            dimension_semantics=("arbitrary",)),
    )(P, E2, WE, bcat, W2, Wa1, ba1, Wa2, ba2, b2, Wu2, bu2)


_SC = 40
_SSUB = _CC // _SC


def _sc_scatter(g, dst1d, zrows):
    M = g.shape[0]
    D = zrows.shape[1]
    Np = zrows.shape[0]
    per_w = M // _NW
    n_ch = per_w // _CC
    rounds = n_ch // 2
    rpt = Np // _NS
    mesh = plsc.VectorSubcoreMesh(core_axis_name="c", subcore_axis_name="s")

    @functools.partial(
        pl.kernel,
        mesh=mesh,
        out_type=jax.ShapeDtypeStruct((2 * Np, D), _F32),
        scratch_types=(
            [pltpu.VMEM((_SC,), jnp.int32) for _ in range(2 * _SSUB)]
            + [
                pltpu.VMEM((_CC, D), _F32),
                pltpu.VMEM((_CC, D), _F32),
                pltpu.VMEM_SHARED((Np, D), _F32),
                pltpu.SemaphoreType.DMA,
                pltpu.SemaphoreType.DMA,
                pltpu.SemaphoreType.DMA,
                pltpu.SemaphoreType.DMA,
            ]
        ),
        compiler_params=pltpu.CompilerParams(use_tc_tiling_on_sc=False),
    )
    def k(g_hbm, dst_hbm, z_hbm, out, *rest):
        idx = [list(rest[:_SSUB]), list(rest[_SSUB:2 * _SSUB])]
        r0, r1, acc, sin0, sin1, ssc0, ssc1 = rest[2 * _SSUB:]
        rows = [r0, r1]
        sem_in = [sin0, sin1]
        sem_sc = [ssc0, ssc1]

        cid = lax.axis_index("c")
        sid = lax.axis_index("s")
        wid = sid * _NC + cid
        base = wid * per_w

        pltpu.sync_copy(z_hbm.at[pl.ds(sid * rpt, rpt)],
                        acc.at[pl.ds(sid * rpt, rpt)])
        plsc.subcore_barrier()

        def fire_in(c, b):
            off = base + c * _CC
            for j in range(_SSUB):
                pltpu.async_copy(dst_hbm.at[pl.ds(off + j * _SC, _SC)],
                                 idx[b][j], sem_in[b])
            pltpu.async_copy(g_hbm.at[pl.ds(off, _CC), pl.ds(0, D)], rows[b],
                             sem_in[b])

        def wait_in(b):
            for j in range(_SSUB):
                pltpu.make_async_copy(dst_hbm.at[pl.ds(0, _SC)], idx[b][j],
                                      sem_in[b]).wait()
            pltpu.make_async_copy(g_hbm.at[pl.ds(0, _CC), pl.ds(0, D)],
                                  rows[b], sem_in[b]).wait()

        def fire_scatter(b):
            for j in range(_SSUB):
                pltpu.async_copy(rows[b].at[pl.ds(j * _SC, _SC)],
                                 acc.at[idx[b][j]], sem_sc[b], add=True)

        def wait_scatter(b):
            for j in range(_SSUB):
                pltpu.make_async_copy(rows[b].at[pl.ds(j * _SC, _SC)],
                                      acc.at[idx[b][j]], sem_sc[b]).wait()

        for b in range(2):
            fire_in(b, b)

        def round_body(r, carry):
            for b in range(2):
                wait_in(b)
                fire_scatter(b)
            for b in range(2):
                wait_scatter(b)
                fire_in(2 * r + b + 2, b)
            return carry

        lax.fori_loop(0, rounds - 1, round_body, 0)

        for b in range(2):
            wait_in(b)
            fire_scatter(b)
        for b in range(2):
            wait_scatter(b)

        plsc.subcore_barrier()
        pltpu.sync_copy(acc.at[pl.ds(sid * rpt, rpt)],
                        out.at[pl.ds(cid * Np + sid * rpt, rpt)])

    return k(g, dst1d, zrows)


def _tc_nodes(V2, G0, G1, Wn1, bn1, Wn2, bn2, W2, b2):
    N = V2.shape[0]

    def body(v_ref, g0_ref, g1_ref, wn1_ref, bn1_ref,
             wn2_ref, bn2_ref, w2_ref, b2_ref, out_ref):
        gs = g0_ref[...] + g1_ref[...]
        Gm = gs[:, :32]
        dn = gs[:, 32:33]
        u = Gm / (dn + 1e-16)
        wn1a = wn1_ref[:128, :]
        wn1b = wn1_ref[128:, :]
        w2n = jnp.dot(w2_ref[...], wn1b, preferred_element_type=_F32)
        bw = jnp.dot(b2_ref[...], wn1b, preferred_element_type=_F32)
        pre = (jnp.dot(v_ref[...], wn1a, preferred_element_type=_F32)
               + jnp.dot(u, w2n, preferred_element_type=_F32)
               + bn1_ref[...]
               + jnp.where(dn > 0, bw, 0.0))
        out_ref[...] = (jnp.dot(_silu(pre), wn2_ref[...],
                                preferred_element_type=_F32) + bn2_ref[...])

    return pl.pallas_call(
        body,
        out_shape=jax.ShapeDtypeStruct((N, 128), _F32),
    )(V2, G0, G1, Wn1, bn1, Wn2, bn2, W2, b2)


def kernel(V, E, edges, W1, b1, W2, b2, Wa1, ba1, Wa2, ba2, Wu1, bu1, Wu2, bu2,
           Wn1, bn1, Wn2, bn2):
    V2 = V[0]
    E2 = E[0]
    M = E2.shape[0]
    src2d = edges[0, :, 0].reshape(M // _C, _C)
    dst1d = edges[0, :, 1]
    dst2d = dst1d.reshape(M // _C, _C)
    N = V2.shape[0]

    WA = jnp.concatenate([W1[:128], Wu1[:128]], axis=1)
    WB = jnp.concatenate([W1[128:256], Wu1[128:256]], axis=1)
    WE = jnp.concatenate([W1[256:], Wu1[256:]], axis=1)
    bcat = jnp.concatenate([b1, bu1])[None, :]

    A, B = _tc_tables(V2, WA, WB)
    P = _sc_gather(A, B, src2d, dst2d, M)
    delta, g = _tc_edges(P, E2, WE, bcat, W2, Wa1, ba1[None, :], Wa2,
                         ba2[None, :], b2[None, :], Wu2, bu2[None, :])
    Np = 16 * 640
    zrows = jnp.zeros((Np, 48), _F32)
    Gp = _sc_scatter(g, dst1d, zrows)
    node_out = _tc_nodes(V2, Gp[:N], Gp[Np:Np + N], Wn1, bn1[None, :], Wn2,
                         bn2[None, :], W2, b2[None, :])
    return node_out[None], delta[None]


# speedup vs baseline: 1.3788x; 1.0369x over previous
import functools

import jax
import jax.numpy as jnp
from jax import lax
from jax.experimental import pallas as pl
from jax.experimental.pallas import tpu as pltpu
from jax.experimental.pallas import tpu_sc as plsc

_F32 = jnp.float32
_NC, _NS = 2, 16
_NW = _NC * _NS
_C = 100
_SUB = 2
_CC = _C * _SUB


def _silu(x):
    return x * jax.nn.sigmoid(x)


def _tc_tables(V2, WA, WB):
    N = V2.shape[0]

    def body(v_ref, wa_ref, wb_ref, a_ref, b_ref):
        v = v_ref[...]
        a_ref[...] = jnp.dot(v, wa_ref[...], preferred_element_type=_F32)
        b_ref[...] = jnp.dot(v, wb_ref[...], preferred_element_type=_F32)

    return pl.pallas_call(
        body,
        out_shape=[
            jax.ShapeDtypeStruct((N, 64), _F32),
            jax.ShapeDtypeStruct((N, 64), _F32),
        ],
    )(V2, WA, WB)


def _sc_gather(A, B, src2d, dst2d, M):
    D = A.shape[1]
    per_w = M // _NW
    n_ch = per_w // _CC
    rounds = n_ch // 2
    mesh = plsc.VectorSubcoreMesh(core_axis_name="c", subcore_axis_name="s")

    @functools.partial(
        pl.kernel,
        mesh=mesh,
        out_type=jax.ShapeDtypeStruct((M, 2 * D), _F32),
        scratch_types=[
            pltpu.VMEM((_SUB, _C), jnp.int32),
            pltpu.VMEM((_SUB, _C), jnp.int32),
            pltpu.VMEM((_SUB, _C), jnp.int32),
            pltpu.VMEM((_SUB, _C), jnp.int32),
            pltpu.VMEM((_CC, D), _F32),
            pltpu.VMEM((_CC, D), _F32),
            pltpu.VMEM((_CC, D), _F32),
            pltpu.VMEM((_CC, D), _F32),
            pltpu.SemaphoreType.DMA,
            pltpu.SemaphoreType.DMA,
            pltpu.SemaphoreType.DMA,
            pltpu.SemaphoreType.DMA,
            pltpu.SemaphoreType.DMA,
            pltpu.SemaphoreType.DMA,
        ],
        compiler_params=pltpu.CompilerParams(use_tc_tiling_on_sc=False),
    )
    def k(a_hbm, b_hbm, src_hbm, dst_hbm, outp, idxs0, idxs1, idxd0, idxd1,
          ga0, ga1, gb0, gb1, sg0, sg1, sw0, sw1, si0, si1):
        idxs = [idxs0, idxs1]
        idxd = [idxd0, idxd1]
        ga = [ga0, ga1]
        gb = [gb0, gb1]
        sem_g = [sg0, sg1]
        sem_w = [sw0, sw1]
        sem_i = [si0, si1]

        wid = lax.axis_index("s") * _NC + lax.axis_index("c")
        base = wid * per_w
        idx_base = wid * (per_w // _C)

        def fire_idx(c, b):
            r0 = idx_base + c * _SUB
            pltpu.async_copy(src_hbm.at[pl.ds(r0, _SUB)], idxs[b], sem_i[b])
            pltpu.async_copy(dst_hbm.at[pl.ds(r0, _SUB)], idxd[b], sem_i[b])

        def wait_idx(b):
            pltpu.make_async_copy(src_hbm.at[pl.ds(0, _SUB)], idxs[b],
                                  sem_i[b]).wait()
            pltpu.make_async_copy(dst_hbm.at[pl.ds(0, _SUB)], idxd[b],
                                  sem_i[b]).wait()

        def fire_gather(b):
            for j in range(_SUB):
                pltpu.async_copy(a_hbm.at[idxs[b].at[j]],
                                 ga[b].at[pl.ds(j * _C, _C)], sem_g[b])
                pltpu.async_copy(b_hbm.at[idxd[b].at[j]],
                                 gb[b].at[pl.ds(j * _C, _C)], sem_g[b])

        def wait_gather(b):
            for j in range(_SUB):
                pltpu.make_async_copy(a_hbm.at[idxs[b].at[j]],
                                      ga[b].at[pl.ds(j * _C, _C)],
                                      sem_g[b]).wait()
                pltpu.make_async_copy(b_hbm.at[idxd[b].at[j]],
                                      gb[b].at[pl.ds(j * _C, _C)],
                                      sem_g[b]).wait()

        def fire_write(c, b):
            sl = pl.ds(base + c * _CC, _CC)
            pltpu.async_copy(ga[b], outp.at[sl, pl.ds(0, D)], sem_w[b])
            pltpu.async_copy(gb[b], outp.at[sl, pl.ds(D, D)], sem_w[b])

        def wait_write(b):
            sl = pl.ds(0, _CC)
            pltpu.make_async_copy(ga[b], outp.at[sl, pl.ds(0, D)],
                                  sem_w[b]).wait()
            pltpu.make_async_copy(gb[b], outp.at[sl, pl.ds(D, D)],
                                  sem_w[b]).wait()

        for b in range(2):
            fire_idx(b, b)
        for b in range(2):
            wait_idx(b)
            fire_gather(b)

        for b in range(2):
            wait_gather(b)
            fire_write(b, b)
            fire_idx(b + 2, b)
        for b in range(2):
            wait_idx(b)
            wait_write(b)
            fire_gather(b)

        def round_body(r, carry):
            for b in range(2):
                c = 2 * r + b
                wait_gather(b)
                fire_write(c, b)
                fire_idx(c + 2, b)
            for b in range(2):
                wait_idx(b)
                wait_write(b)
                fire_gather(b)
            return carry

        lax.fori_loop(1, rounds - 1, round_body, 0)

        for b in range(2):
            wait_gather(b)
            fire_write(n_ch - 2 + b, b)
        for b in range(2):
            wait_write(b)

    return k(A, B, src2d, dst2d)


def _tc_edges(P, E2, WE, bcat, W2, Wa1, ba1, Wa2, ba2, b2, Wu2, bu2):
    M = E2.shape[0]
    Mb = 8000
    grid = (M // Mb,)

    def body(p_ref, e_ref, we_ref, bc_ref, w2_ref, wa1_ref, ba1_ref,
             wa2_ref, ba2_ref, b2_ref, wu2_ref, bu2_ref, delta_ref, g_ref):
        pp = p_ref[...]
        h = (pp[:, :64] + pp[:, 64:]
             + jnp.dot(e_ref[...], we_ref[...], preferred_element_type=_F32)
             + bc_ref[...])
        s = _silu(h)
        s1 = s[:, :32]
        su = s[:, 32:]
        w2a = jnp.dot(w2_ref[...], wa1_ref[...], preferred_element_type=_F32)
        ba1p = ba1_ref[...] + jnp.dot(b2_ref[...], wa1_ref[...],
                                      preferred_element_type=_F32)
        t = _silu(jnp.dot(s1, w2a, preferred_element_type=_F32) + ba1p)
        logit = jnp.dot(t, wa2_ref[...], preferred_element_type=_F32) + ba2_ref[...]
        w = jnp.exp(jnp.clip(logit, -30.0, 30.0))
        g_ref[...] = jnp.concatenate(
            [w * s1, jnp.broadcast_to(w, (Mb, 16)),
             jnp.zeros((Mb, 80), _F32)], axis=1)
        delta_ref[...] = (jnp.dot(su, wu2_ref[...], preferred_element_type=_F32)
                          + bu2_ref[...])

    full = lambda shape: pl.BlockSpec(shape, lambda i: (0, 0))
    return pl.pallas_call(
        body,
        grid=grid,
        in_specs=[
            pl.BlockSpec((Mb, 128), lambda i: (i, 0)),
            pl.BlockSpec((Mb, 128), lambda i: (i, 0)),
            full((128, 64)),
            full((1, 64)),
            full((32, 128)),
            full((128, 8)),
            full((1, 8)),
            full((8, 1)),
            full((1, 1)),
            full((1, 128)),
            full((32, 128)),
            full((1, 128)),
        ],
        out_specs=[
            pl.BlockSpec((Mb, 128), lambda i: (i, 0)),
            pl.BlockSpec((Mb, 128), lambda i: (i, 0)),
        ],
        out_shape=[
            jax.ShapeDtypeStruct((M, 128), _F32),
            jax.ShapeDtypeStruct((M, 128), _F32),
        ],
        compiler_params=pltpu.CompilerParams(
            dimension_semantics=("arbitrary",)),
    )(P, E2, WE, bcat, W2, Wa1, ba1, Wa2, ba2, b2, Wu2, bu2)


_SC = 40
_SSUB = _CC // _SC


def _sc_scatter(g, dst1d, zrows):
    M = g.shape[0]
    D = zrows.shape[1]
    Np = zrows.shape[0]
    per_w = M // _NW
    n_ch = per_w // _CC
    rounds = n_ch // 2
    rpt = Np // _NS
    mesh = plsc.VectorSubcoreMesh(core_axis_name="c", subcore_axis_name="s")

    @functools.partial(
        pl.kernel,
        mesh=mesh,
        out_type=jax.ShapeDtypeStruct((2 * Np, D), _F32),
        scratch_types=(
            [pltpu.VMEM((_SC,), jnp.int32) for _ in range(2 * _SSUB)]
            + [
                pltpu.VMEM((_CC, D), _F32),
                pltpu.VMEM((_CC, D), _F32),
                pltpu.VMEM_SHARED((Np, D), _F32),
                pltpu.SemaphoreType.DMA,
                pltpu.SemaphoreType.DMA,
                pltpu.SemaphoreType.DMA,
                pltpu.SemaphoreType.DMA,
            ]
        ),
        compiler_params=pltpu.CompilerParams(use_tc_tiling_on_sc=False),
    )
    def k(g_hbm, dst_hbm, z_hbm, out, *rest):
        idx = [list(rest[:_SSUB]), list(rest[_SSUB:2 * _SSUB])]
        r0, r1, acc, sin0, sin1, ssc0, ssc1 = rest[2 * _SSUB:]
        rows = [r0, r1]
        sem_in = [sin0, sin1]
        sem_sc = [ssc0, ssc1]

        cid = lax.axis_index("c")
        sid = lax.axis_index("s")
        wid = sid * _NC + cid
        base = wid * per_w

        pltpu.sync_copy(z_hbm.at[pl.ds(sid * rpt, rpt)],
                        acc.at[pl.ds(sid * rpt, rpt)])
        plsc.subcore_barrier()

        def fire_in(c, b):
            off = base + c * _CC
            for j in range(_SSUB):
                pltpu.async_copy(dst_hbm.at[pl.ds(off + j * _SC, _SC)],
                                 idx[b][j], sem_in[b])
            pltpu.async_copy(g_hbm.at[pl.ds(off, _CC), pl.ds(0, D)], rows[b],
                             sem_in[b])

        def wait_in(b):
            for j in range(_SSUB):
                pltpu.make_async_copy(dst_hbm.at[pl.ds(0, _SC)], idx[b][j],
                                      sem_in[b]).wait()
            pltpu.make_async_copy(g_hbm.at[pl.ds(0, _CC), pl.ds(0, D)],
                                  rows[b], sem_in[b]).wait()

        def fire_scatter(b):
            for j in range(_SSUB):
                pltpu.async_copy(rows[b].at[pl.ds(j * _SC, _SC)],
                                 acc.at[idx[b][j]], sem_sc[b], add=True)

        def wait_scatter(b):
            for j in range(_SSUB):
                pltpu.make_async_copy(rows[b].at[pl.ds(j * _SC, _SC)],
                                      acc.at[idx[b][j]], sem_sc[b]).wait()

        for b in range(2):
            fire_in(b, b)

        def round_body(r, carry):
            for b in range(2):
                wait_in(b)
                fire_scatter(b)
            for b in range(2):
                wait_scatter(b)
                fire_in(2 * r + b + 2, b)
            return carry

        lax.fori_loop(0, rounds - 1, round_body, 0)

        for b in range(2):
            wait_in(b)
            fire_scatter(b)
        for b in range(2):
            wait_scatter(b)

        plsc.subcore_barrier()
        pltpu.sync_copy(acc.at[pl.ds(sid * rpt, rpt)],
                        out.at[pl.ds(cid * Np + sid * rpt, rpt)])

    return k(g, dst1d, zrows)


def _tc_nodes(V2, G0, G1, Wn1, bn1, Wn2, bn2, W2, b2):
    N = V2.shape[0]

    def body(v_ref, g0_ref, g1_ref, wn1_ref, bn1_ref,
             wn2_ref, bn2_ref, w2_ref, b2_ref, out_ref):
        gs = g0_ref[...] + g1_ref[...]
        Gm = gs[:, :32]
        dn = gs[:, 32:33]
        u = Gm / (dn + 1e-16)
        wn1a = wn1_ref[:128, :]
        wn1b = wn1_ref[128:, :]
        w2n = jnp.dot(w2_ref[...], wn1b, preferred_element_type=_F32)
        bw = jnp.dot(b2_ref[...], wn1b, preferred_element_type=_F32)
        pre = (jnp.dot(v_ref[...], wn1a, preferred_element_type=_F32)
               + jnp.dot(u, w2n, preferred_element_type=_F32)
               + bn1_ref[...]
               + jnp.where(dn > 0, bw, 0.0))
        out_ref[...] = (jnp.dot(_silu(pre), wn2_ref[...],
                                preferred_element_type=_F32) + bn2_ref[...])

    return pl.pallas_call(
        body,
        out_shape=jax.ShapeDtypeStruct((N, 128), _F32),
    )(V2, G0, G1, Wn1, bn1, Wn2, bn2, W2, b2)


def kernel(V, E, edges, W1, b1, W2, b2, Wa1, ba1, Wa2, ba2, Wu1, bu1, Wu2, bu2,
           Wn1, bn1, Wn2, bn2):
    V2 = V[0]
    E2 = E[0]
    M = E2.shape[0]
    src2d = edges[0, :, 0].reshape(M // _C, _C)
    dst1d = edges[0, :, 1]
    dst2d = dst1d.reshape(M // _C, _C)
    N = V2.shape[0]

    WA = jnp.concatenate([W1[:128], Wu1[:128]], axis=1)
    WB = jnp.concatenate([W1[128:256], Wu1[128:256]], axis=1)
    WE = jnp.concatenate([W1[256:], Wu1[256:]], axis=1)
    bcat = jnp.concatenate([b1, bu1])[None, :]

    A, B = _tc_tables(V2, WA, WB)
    P = _sc_gather(A, B, src2d, dst2d, M)
    delta, g = _tc_edges(P, E2, WE, bcat, W2, Wa1, ba1[None, :], Wa2,
                         ba2[None, :], b2[None, :], Wu2, bu2[None, :])
    Np = 16 * 640
    zrows = jnp.zeros((Np, 48), _F32)
    Gp = _sc_scatter(g, dst1d, zrows)
    node_out = _tc_nodes(V2, Gp[:N], Gp[Np:Np + N], Wn1, bn1[None, :], Wn2,
                         bn2[None, :], W2, b2[None, :])
    return node_out[None], delta[None]
